# baseline (speedup 1.0000x reference)
import functools

import jax
import jax.numpy as jnp
from jax import lax
from jax.experimental import pallas as pl
from jax.experimental.pallas import tpu as pltpu
from jax.experimental.pallas import tpu_sc as plsc

_N_NODES = 100000
_LANES = 16
_NC = 2
_NS = 16
_NW = _NC * _NS
_CHUNK = 3200
_UNROLL = 10
_NPAD = 100352


def _make_sc_histogram(e):
    nchunks = e // _CHUNK
    assert nchunks * _CHUNK == e and _CHUNK % 128 == 0
    assert _CHUNK % (_LANES * _UNROLL) == 0
    nrounds = nchunks // _NW
    nleft = nchunks - nrounds * _NW
    npairs = nrounds // 2
    assert npairs * 2 == nrounds and nleft < _NW

    mesh = plsc.VectorSubcoreMesh(core_axis_name="c", subcore_axis_name="s")

    @functools.partial(
        pl.kernel,
        out_type=jax.ShapeDtypeStruct((_NW, _NPAD), jnp.float32),
        mesh=mesh,
        compiler_params=pltpu.CompilerParams(needs_layout_passes=False),
        scratch_types=[
            pltpu.VMEM((_NPAD,), jnp.float32),
            pltpu.VMEM((2, _CHUNK), jnp.int32),
            pltpu.VMEM((2, _CHUNK), jnp.int32),
            pltpu.SemaphoreType.DMA,
            pltpu.SemaphoreType.DMA,
            pltpu.SemaphoreType.DMA,
        ],
    )
    def hist_kernel(src_hbm, zeros_hbm, out_hbm, hist, ebuf0, ebuf1, sem0,
                    sem1, semz):
        cid = lax.axis_index("c")
        sid = lax.axis_index("s")
        wid = cid * _NS + sid

        ones = jnp.ones((_LANES,), jnp.float32)

        def start(chunk, buf, sem):
            off = pl.multiple_of(chunk * _CHUNK, 128)
            pltpu.async_copy(src_hbm.at[:, pl.ds(off, _CHUNK)], buf, sem)

        def wait(buf, sem):
            pltpu.make_async_copy(src_hbm.at[:, pl.ds(0, _CHUNK)], buf,
                                  sem).wait()

        def scatter(buf):
            @plsc.parallel_loop(0, _CHUNK // _LANES, 1, unroll=_UNROLL)
            def _(j):
                idx = buf[0, pl.ds(j * _LANES, _LANES)]
                plsc.addupdate_scatter(hist, [idx], ones)

        pltpu.async_copy(zeros_hbm, hist, semz)
        start(wid, ebuf0, sem0)
        pltpu.make_async_copy(zeros_hbm, hist, semz).wait()

        def pair_body(c, _):
            start(wid + (2 * c + 1) * _NW, ebuf1, sem1)
            wait(ebuf0, sem0)
            scatter(ebuf0)

            @pl.when(c < npairs - 1)
            def _():
                start(wid + (2 * c + 2) * _NW, ebuf0, sem0)

            @pl.when((c == npairs - 1) & (wid < nleft))
            def _():
                start(nrounds * _NW + wid, ebuf0, sem0)

            wait(ebuf1, sem1)
            scatter(ebuf1)
            return 0

        lax.fori_loop(0, npairs, pair_body, 0)

        @pl.when(wid < nleft)
        def _():
            wait(ebuf0, sem0)
            scatter(ebuf0)

        pltpu.sync_copy(hist, out_hbm.at[wid])

    return hist_kernel


_FBLK = 6272


def _combine(nn, partials):
    def fin(nn_ref, p_ref, o_ref):
        i = pl.program_id(0)
        total = jnp.sum(p_ref[...], axis=0, keepdims=True)
        col = lax.broadcasted_iota(jnp.int32, (1, _FBLK), 1) + i * _FBLK
        o_ref[...] = jnp.where(col < nn_ref[0], total, jnp.float32(0.0))

    return pl.pallas_call(
        fin,
        grid=(_NPAD // _FBLK,),
        out_shape=jax.ShapeDtypeStruct((1, _NPAD), jnp.float32),
        in_specs=[
            pl.BlockSpec(memory_space=pltpu.SMEM),
            pl.BlockSpec((_NW, _FBLK), lambda i: (0, i)),
        ],
        out_specs=pl.BlockSpec((1, _FBLK), lambda i: (0, i)),
    )(nn, partials)


def kernel(edge_index, num_nodes):
    e = edge_index.shape[1]
    src = edge_index.astype(jnp.int32)
    zeros = jnp.zeros((_NPAD,), jnp.float32)

    partials = _make_sc_histogram(e)(src, zeros)
    nn = jnp.asarray(num_nodes, jnp.int32).reshape(1)
    deg = _combine(nn, partials)
    return deg.reshape(-1)[:_N_NODES][:, None]


# speedup vs baseline: 1.0820x; 1.0820x over previous
import functools

import jax
import jax.numpy as jnp
from jax import lax
from jax.experimental import pallas as pl
from jax.experimental.pallas import tpu as pltpu
from jax.experimental.pallas import tpu_sc as plsc

_N_NODES = 100000
_LANES = 16
_NC = 2
_NS = 16
_NW = _NC * _NS
_CHUNK = 3200
_UNROLL = 10
_NPAD = 100352


def _make_sc_histogram(e):
    nchunks = e // _CHUNK
    assert nchunks * _CHUNK == e and _CHUNK % 128 == 0
    assert _CHUNK % (_LANES * _UNROLL) == 0
    nrounds = nchunks // _NW
    nleft = nchunks - nrounds * _NW
    npairs = nrounds // 2
    assert npairs * 2 == nrounds and nleft < _NW

    mesh = plsc.VectorSubcoreMesh(core_axis_name="c", subcore_axis_name="s")

    @functools.partial(
        pl.kernel,
        out_type=jax.ShapeDtypeStruct((_NW, _NPAD), jnp.float32),
        mesh=mesh,
        compiler_params=pltpu.CompilerParams(needs_layout_passes=False),
        scratch_types=[
            pltpu.VMEM((_NPAD,), jnp.float32),
            pltpu.VMEM((2, _CHUNK), jnp.int32),
            pltpu.VMEM((2, _CHUNK), jnp.int32),
            pltpu.SemaphoreType.DMA,
            pltpu.SemaphoreType.DMA,
        ],
    )
    def hist_kernel(src_hbm, out_hbm, hist, ebuf0, ebuf1, sem0, sem1):
        cid = lax.axis_index("c")
        sid = lax.axis_index("s")
        wid = cid * _NS + sid

        ones = jnp.ones((_LANES,), jnp.float32)

        def start(chunk, buf, sem):
            off = pl.multiple_of(chunk * _CHUNK, 128)
            pltpu.async_copy(src_hbm.at[:, pl.ds(off, _CHUNK)], buf, sem)

        def wait(buf, sem):
            pltpu.make_async_copy(src_hbm.at[:, pl.ds(0, _CHUNK)], buf,
                                  sem).wait()

        def scatter(buf):
            @plsc.parallel_loop(0, _CHUNK // _LANES, 1, unroll=_UNROLL)
            def _(j):
                idx = buf[0, pl.ds(j * _LANES, _LANES)]
                plsc.addupdate_scatter(hist, [idx], ones)

        start(wid, ebuf0, sem0)

        @plsc.parallel_loop(0, _NPAD // _LANES, 1, unroll=16)
        def _(i):
            hist[pl.ds(i * _LANES, _LANES)] = jnp.zeros((_LANES,), jnp.float32)

        def pair_body(c, _):
            start(wid + (2 * c + 1) * _NW, ebuf1, sem1)
            wait(ebuf0, sem0)
            scatter(ebuf0)

            @pl.when(c < npairs - 1)
            def _():
                start(wid + (2 * c + 2) * _NW, ebuf0, sem0)

            @pl.when((c == npairs - 1) & (wid < nleft))
            def _():
                start(nrounds * _NW + wid, ebuf0, sem0)

            wait(ebuf1, sem1)
            scatter(ebuf1)
            return 0

        lax.fori_loop(0, npairs, pair_body, 0)

        @pl.when(wid < nleft)
        def _():
            wait(ebuf0, sem0)
            scatter(ebuf0)

        pltpu.sync_copy(hist, out_hbm.at[wid])

    return hist_kernel


_FBLK = 6272


def _combine(nn, partials):
    def fin(nn_ref, p_ref, o_ref):
        i = pl.program_id(0)
        total = jnp.sum(p_ref[...], axis=0, keepdims=True)
        col = lax.broadcasted_iota(jnp.int32, (1, _FBLK), 1) + i * _FBLK
        o_ref[...] = jnp.where(col < nn_ref[0], total, jnp.float32(0.0))

    return pl.pallas_call(
        fin,
        grid=(_NPAD // _FBLK,),
        out_shape=jax.ShapeDtypeStruct((1, _NPAD), jnp.float32),
        in_specs=[
            pl.BlockSpec(memory_space=pltpu.SMEM),
            pl.BlockSpec((_NW, _FBLK), lambda i: (0, i)),
        ],
        out_specs=pl.BlockSpec((1, _FBLK), lambda i: (0, i)),
    )(nn, partials)


def kernel(edge_index, num_nodes):
    e = edge_index.shape[1]
    src = edge_index.astype(jnp.int32)

    partials = _make_sc_histogram(e)(src)
    nn = jnp.asarray(num_nodes, jnp.int32).reshape(1)
    deg = _combine(nn, partials)
    return deg.reshape(-1)[:_N_NODES][:, None]


# speedup vs baseline: 1.1329x; 1.0470x over previous
import functools

import jax
import jax.numpy as jnp
from jax import lax
from jax.experimental import pallas as pl
from jax.experimental.pallas import tpu as pltpu
from jax.experimental.pallas import tpu_sc as plsc

_N_NODES = 100000
_LANES = 16
_NC = 2
_NS = 16
_NW = _NC * _NS
_CHUNK = 3200
_UNROLL = 10
_NPAD = 100352


def _make_sc_histogram(e):
    nchunks = e // _CHUNK
    assert nchunks * _CHUNK == e and _CHUNK % 128 == 0
    assert _CHUNK % (_LANES * _UNROLL) == 0
    nrounds = nchunks // _NW
    nleft = nchunks - nrounds * _NW
    npairs = nrounds // 2
    assert npairs * 2 == nrounds and nleft < _NW

    mesh = plsc.VectorSubcoreMesh(core_axis_name="c", subcore_axis_name="s")

    @functools.partial(
        pl.kernel,
        out_type=jax.ShapeDtypeStruct((_NW, _NPAD), jnp.float32),
        mesh=mesh,
        compiler_params=pltpu.CompilerParams(needs_layout_passes=False),
        scratch_types=[
            pltpu.VMEM((_NPAD,), jnp.float32),
            pltpu.VMEM((2, _CHUNK), jnp.int32),
            pltpu.VMEM((2, _CHUNK), jnp.int32),
            pltpu.SemaphoreType.DMA,
            pltpu.SemaphoreType.DMA,
        ],
    )
    def hist_kernel(src_hbm, out_hbm, hist, ebuf0, ebuf1, sem0, sem1):
        cid = lax.axis_index("c")
        sid = lax.axis_index("s")
        wid = cid * _NS + sid

        ones = jnp.ones((_LANES,), jnp.float32)

        def start(chunk, buf, sem):
            off = pl.multiple_of(chunk * _CHUNK, 128)
            pltpu.async_copy(src_hbm.at[:, pl.ds(off, _CHUNK)], buf, sem)

        def wait(buf, sem):
            pltpu.make_async_copy(src_hbm.at[:, pl.ds(0, _CHUNK)], buf,
                                  sem).wait()

        def scatter(buf):
            @plsc.parallel_loop(0, _CHUNK // _LANES, 1, unroll=_UNROLL)
            def _(j):
                idx = buf[0, pl.ds(j * _LANES, _LANES)]
                plsc.addupdate_scatter(hist, [idx], ones)

        start(wid, ebuf0, sem0)

        @plsc.parallel_loop(0, _NPAD // _LANES, 1, unroll=16)
        def _(i):
            hist[pl.ds(i * _LANES, _LANES)] = jnp.zeros((_LANES,), jnp.float32)

        def pair_body(c, _):
            start(wid + (2 * c + 1) * _NW, ebuf1, sem1)
            wait(ebuf0, sem0)
            scatter(ebuf0)

            @pl.when(c < npairs - 1)
            def _():
                start(wid + (2 * c + 2) * _NW, ebuf0, sem0)

            @pl.when((c == npairs - 1) & (wid < nleft))
            def _():
                start(nrounds * _NW + wid, ebuf0, sem0)

            wait(ebuf1, sem1)
            scatter(ebuf1)
            return 0

        lax.fori_loop(0, npairs, pair_body, 0)

        @pl.when(wid < nleft)
        def _():
            wait(ebuf0, sem0)
            scatter(ebuf0)

        pltpu.sync_copy(hist, out_hbm.at[wid])

    return hist_kernel


_FBLK = 12544


def _combine(nn, partials):
    def fin(nn_ref, p_ref, o_ref):
        i = pl.program_id(0)
        total = jnp.sum(p_ref[...], axis=0, keepdims=True)
        col = lax.broadcasted_iota(jnp.int32, (1, _FBLK), 1) + i * _FBLK
        o_ref[...] = jnp.where(col < nn_ref[0], total, jnp.float32(0.0))

    return pl.pallas_call(
        fin,
        grid=(_NPAD // _FBLK,),
        out_shape=jax.ShapeDtypeStruct((1, _NPAD), jnp.float32),
        in_specs=[
            pl.BlockSpec(memory_space=pltpu.SMEM),
            pl.BlockSpec((_NW, _FBLK), lambda i: (0, i)),
        ],
        out_specs=pl.BlockSpec((1, _FBLK), lambda i: (0, i)),
    )(nn, partials)


def kernel(edge_index, num_nodes):
    e = edge_index.shape[1]
    src = edge_index.astype(jnp.int32)

    partials = _make_sc_histogram(e)(src)
    nn = jnp.asarray(num_nodes, jnp.int32).reshape(1)
    deg = _combine(nn, partials)
    return deg.reshape(-1)[:_N_NODES][:, None]


# speedup vs baseline: 1.1585x; 1.0226x over previous
import functools

import jax
import jax.numpy as jnp
from jax import lax
from jax.experimental import pallas as pl
from jax.experimental.pallas import tpu as pltpu
from jax.experimental.pallas import tpu_sc as plsc

_N_NODES = 100000
_LANES = 16
_NC = 2
_NS = 16
_NW = _NC * _NS
_CHUNK = 3200
_UNROLL = 10
_NPAD = 100352


def _make_sc_histogram(e):
    nchunks = e // _CHUNK
    assert nchunks * _CHUNK == e and _CHUNK % 128 == 0
    assert _CHUNK % (_LANES * _UNROLL) == 0
    nrounds = nchunks // _NW
    nleft = nchunks - nrounds * _NW
    npairs = nrounds // 2
    assert npairs * 2 == nrounds and nleft < _NW

    mesh = plsc.VectorSubcoreMesh(core_axis_name="c", subcore_axis_name="s")

    @functools.partial(
        pl.kernel,
        out_type=jax.ShapeDtypeStruct((_NW, _NPAD), jnp.float32),
        mesh=mesh,
        compiler_params=pltpu.CompilerParams(needs_layout_passes=False),
        scratch_types=[
            pltpu.VMEM((_NPAD,), jnp.float32),
            pltpu.VMEM((2, _CHUNK), jnp.int32),
            pltpu.VMEM((2, _CHUNK), jnp.int32),
            pltpu.SemaphoreType.DMA,
            pltpu.SemaphoreType.DMA,
        ],
    )
    def hist_kernel(src_hbm, out_hbm, hist, ebuf0, ebuf1, sem0, sem1):
        cid = lax.axis_index("c")
        sid = lax.axis_index("s")
        wid = cid * _NS + sid

        ones = jnp.ones((_LANES,), jnp.float32)

        def start(chunk, buf, sem):
            off = pl.multiple_of(chunk * _CHUNK, 128)
            pltpu.async_copy(src_hbm.at[:, pl.ds(off, _CHUNK)], buf, sem)

        def wait(buf, sem):
            pltpu.make_async_copy(src_hbm.at[:, pl.ds(0, _CHUNK)], buf,
                                  sem).wait()

        def scatter(buf):
            @plsc.parallel_loop(0, _CHUNK // _LANES, 1, unroll=_UNROLL)
            def _(j):
                idx = buf[0, pl.ds(j * _LANES, _LANES)]
                plsc.addupdate_scatter(hist, [idx], ones)

        start(wid, ebuf0, sem0)

        @plsc.parallel_loop(0, _NPAD // _LANES, 1, unroll=16)
        def _(i):
            hist[pl.ds(i * _LANES, _LANES)] = jnp.zeros((_LANES,), jnp.float32)

        def pair_body(c, _):
            start(wid + (2 * c + 1) * _NW, ebuf1, sem1)
            wait(ebuf0, sem0)
            scatter(ebuf0)

            @pl.when(c < npairs - 1)
            def _():
                start(wid + (2 * c + 2) * _NW, ebuf0, sem0)

            @pl.when((c == npairs - 1) & (wid < nleft))
            def _():
                start(nrounds * _NW + wid, ebuf0, sem0)

            wait(ebuf1, sem1)
            scatter(ebuf1)
            return 0

        lax.fori_loop(0, npairs, pair_body, 0)

        @pl.when(wid < nleft)
        def _():
            wait(ebuf0, sem0)
            scatter(ebuf0)

        pltpu.sync_copy(hist, out_hbm.at[wid])

    return hist_kernel


_FBLK = 25088


def _combine(nn, partials):
    def fin(nn_ref, p_ref, o_ref):
        i = pl.program_id(0)
        total = jnp.sum(p_ref[...], axis=0, keepdims=True)
        col = lax.broadcasted_iota(jnp.int32, (1, _FBLK), 1) + i * _FBLK
        o_ref[...] = jnp.where(col < nn_ref[0], total, jnp.float32(0.0))

    return pl.pallas_call(
        fin,
        grid=(_NPAD // _FBLK,),
        out_shape=jax.ShapeDtypeStruct((1, _NPAD), jnp.float32),
        in_specs=[
            pl.BlockSpec(memory_space=pltpu.SMEM),
            pl.BlockSpec((_NW, _FBLK), lambda i: (0, i)),
        ],
        out_specs=pl.BlockSpec((1, _FBLK), lambda i: (0, i)),
    )(nn, partials)


def kernel(edge_index, num_nodes):
    e = edge_index.shape[1]
    src = edge_index.astype(jnp.int32)

    partials = _make_sc_histogram(e)(src)
    nn = jnp.asarray(num_nodes, jnp.int32).reshape(1)
    deg = _combine(nn, partials)
    return deg.reshape(-1)[:_N_NODES][:, None]


# speedup vs baseline: 1.1624x; 1.0034x over previous
import functools

import jax
import jax.numpy as jnp
from jax import lax
from jax.experimental import pallas as pl
from jax.experimental.pallas import tpu as pltpu
from jax.experimental.pallas import tpu_sc as plsc

_N_NODES = 100000
_LANES = 16
_NC = 2
_NS = 16
_NW = _NC * _NS
_CHUNK = 3200
_UNROLL = 10
_NPAD = 100352


def _make_sc_histogram(e):
    nchunks = e // _CHUNK
    assert nchunks * _CHUNK == e and _CHUNK % 128 == 0
    assert _CHUNK % (_LANES * _UNROLL) == 0
    nrounds = nchunks // _NW
    nleft = nchunks - nrounds * _NW
    npairs = nrounds // 2
    assert npairs * 2 == nrounds and nleft < _NW

    mesh = plsc.VectorSubcoreMesh(core_axis_name="c", subcore_axis_name="s")

    @functools.partial(
        pl.kernel,
        out_type=jax.ShapeDtypeStruct((_NW, _NPAD), jnp.float32),
        mesh=mesh,
        compiler_params=pltpu.CompilerParams(needs_layout_passes=False),
        scratch_types=[
            pltpu.VMEM((_NPAD,), jnp.float32),
            pltpu.VMEM((2, _CHUNK), jnp.int32),
            pltpu.VMEM((2, _CHUNK), jnp.int32),
            pltpu.SemaphoreType.DMA,
            pltpu.SemaphoreType.DMA,
        ],
    )
    def hist_kernel(src_hbm, out_hbm, hist, ebuf0, ebuf1, sem0, sem1):
        cid = lax.axis_index("c")
        sid = lax.axis_index("s")
        wid = cid * _NS + sid

        ones = jnp.ones((_LANES,), jnp.float32)

        def start(chunk, buf, sem):
            off = pl.multiple_of(chunk * _CHUNK, 128)
            pltpu.async_copy(src_hbm.at[:, pl.ds(off, _CHUNK)], buf, sem)

        def wait(buf, sem):
            pltpu.make_async_copy(src_hbm.at[:, pl.ds(0, _CHUNK)], buf,
                                  sem).wait()

        def scatter(buf):
            @plsc.parallel_loop(0, _CHUNK // _LANES, 1, unroll=_UNROLL)
            def _(j):
                idx = buf[0, pl.ds(j * _LANES, _LANES)]
                plsc.addupdate_scatter(hist, [idx], ones)

        start(wid, ebuf0, sem0)

        @plsc.parallel_loop(0, _NPAD // _LANES, 1, unroll=16)
        def _(i):
            hist[pl.ds(i * _LANES, _LANES)] = jnp.zeros((_LANES,), jnp.float32)

        def pair_body(c, _):
            start(wid + (2 * c + 1) * _NW, ebuf1, sem1)
            wait(ebuf0, sem0)
            scatter(ebuf0)

            @pl.when(c < npairs - 1)
            def _():
                start(wid + (2 * c + 2) * _NW, ebuf0, sem0)

            @pl.when((c == npairs - 1) & (wid < nleft))
            def _():
                start(nrounds * _NW + wid, ebuf0, sem0)

            wait(ebuf1, sem1)
            scatter(ebuf1)
            return 0

        lax.fori_loop(0, npairs, pair_body, 0)

        @pl.when(wid < nleft)
        def _():
            wait(ebuf0, sem0)
            scatter(ebuf0)

        pltpu.sync_copy(hist, out_hbm.at[wid])

    return hist_kernel


_FBLK = 50176


def _combine(nn, partials):
    def fin(nn_ref, p_ref, o_ref):
        i = pl.program_id(0)
        total = jnp.sum(p_ref[...], axis=0, keepdims=True)
        col = lax.broadcasted_iota(jnp.int32, (1, _FBLK), 1) + i * _FBLK
        o_ref[...] = jnp.where(col < nn_ref[0], total, jnp.float32(0.0))

    return pl.pallas_call(
        fin,
        grid=(_NPAD // _FBLK,),
        out_shape=jax.ShapeDtypeStruct((1, _NPAD), jnp.float32),
        in_specs=[
            pl.BlockSpec(memory_space=pltpu.SMEM),
            pl.BlockSpec((_NW, _FBLK), lambda i: (0, i)),
        ],
        out_specs=pl.BlockSpec((1, _FBLK), lambda i: (0, i)),
    )(nn, partials)


def kernel(edge_index, num_nodes):
    e = edge_index.shape[1]
    src = edge_index.astype(jnp.int32)

    partials = _make_sc_histogram(e)(src)
    nn = jnp.asarray(num_nodes, jnp.int32).reshape(1)
    deg = _combine(nn, partials)
    return deg.reshape(-1)[:_N_NODES][:, None]
